# Initial kernel scaffold; baseline (speedup 1.0000x reference)
import jax
import jax.numpy as jnp
from jax.experimental import pallas as pl


def kernel(nodes, first_node_idx, last_node_idx, candidate_indices, W_coord, b_coord, W1_w, W1_b, W2_w, W2_b):
    raise NotImplementedError("write your pallas kernel here")


# speedup vs baseline: 1.2521x; 1.2521x over previous
import jax
import jax.numpy as jnp
from jax.experimental import pallas as pl

B, N, K, D = 1024, 500, 50, 128
NPAD = 512
RPAD = 64
TB = 8


def _kernel(idx_ref, x_ref, y_ref, wrows_ref, w1_ref, w2_ref, out_ref):
    idx = idx_ref[...]
    x = x_ref[...]
    y = y_ref[...]
    cols = jax.lax.broadcasted_iota(jnp.int32, (TB, RPAD, NPAD), 2)
    oh = cols == idx[:, :, None]
    gx = jnp.sum(jnp.where(oh, x[:, None, :], 0.0), axis=2)
    gy = jnp.sum(jnp.where(oh, y[:, None, :], 0.0), axis=2)
    wx = wrows_ref[0, :]
    wy = wrows_ref[1, :]
    bc = wrows_ref[2, :]
    w1b = wrows_ref[3, :]
    w2b = wrows_ref[4, :]
    emb = gx[:, :, None] * wx[None, None, :] \
        + gy[:, :, None] * wy[None, None, :] + bc[None, None, :]
    f = jnp.dot(emb[:, 0, :], w1_ref[...],
                preferred_element_type=jnp.float32) + w1b[None, :]
    l = jnp.dot(emb[:, 1, :], w2_ref[...],
                preferred_element_type=jnp.float32) + w2b[None, :]
    out_ref[...] = jnp.concatenate(
        [f[:, None, :], l[:, None, :], emb[:, 2:RPAD, :]], axis=1)


def kernel(nodes, first_node_idx, last_node_idx, candidate_indices,
           W_coord, b_coord, W1_w, W1_b, W2_w, W2_b):
    x = jnp.pad(nodes[:, :, 0], ((0, 0), (0, NPAD - N)))
    y = jnp.pad(nodes[:, :, 1], ((0, 0), (0, NPAD - N)))
    idx = jnp.concatenate(
        [first_node_idx[:, None], last_node_idx[:, None],
         jnp.clip(candidate_indices, 0, None)], axis=1).astype(jnp.int32)
    idx = jnp.pad(idx, ((0, 0), (0, RPAD - (2 + K))),
                  constant_values=NPAD - 1)
    wrows = jnp.zeros((8, D), jnp.float32)
    wrows = wrows.at[0].set(W_coord[0]).at[1].set(W_coord[1])
    wrows = wrows.at[2].set(b_coord).at[3].set(W1_b).at[4].set(W2_b)

    grid = (B // TB,)
    out = pl.pallas_call(
        _kernel,
        grid=grid,
        in_specs=[
            pl.BlockSpec((TB, RPAD), lambda i: (i, 0)),
            pl.BlockSpec((TB, NPAD), lambda i: (i, 0)),
            pl.BlockSpec((TB, NPAD), lambda i: (i, 0)),
            pl.BlockSpec((8, D), lambda i: (0, 0)),
            pl.BlockSpec((D, D), lambda i: (0, 0)),
            pl.BlockSpec((D, D), lambda i: (0, 0)),
        ],
        out_specs=pl.BlockSpec((TB, RPAD, D), lambda i: (i, 0, 0)),
        out_shape=jax.ShapeDtypeStruct((B, RPAD, D), jnp.float32),
    )(idx, x, y, wrows, W1_w, W2_w)
    return out[:, : 2 + K, :]


# speedup vs baseline: 1.6257x; 1.2983x over previous
import jax
import jax.numpy as jnp
from jax.experimental import pallas as pl

B, N, K, D = 1024, 500, 50, 128
R = 2 + K
RPAD = 64
TB = 8


def _kernel(idx_ref, x_ref, y_ref, wrows_ref, w1_ref, w2_ref, out_ref):
    idx = idx_ref[...]
    x = x_ref[...]
    y = y_ref[...]
    cols = jax.lax.broadcasted_iota(jnp.int32, (TB, RPAD, N), 2)
    oh = cols == idx[:, :, None]
    gx = jnp.sum(jnp.where(oh, x[:, None, :], 0.0), axis=2)
    gy = jnp.sum(jnp.where(oh, y[:, None, :], 0.0), axis=2)
    wx = wrows_ref[0, :]
    wy = wrows_ref[1, :]
    bc = wrows_ref[2, :]
    w1b = wrows_ref[3, :]
    w2b = wrows_ref[4, :]
    emb = gx[:, :, None] * wx[None, None, :] \
        + gy[:, :, None] * wy[None, None, :] + bc[None, None, :]
    f = jnp.dot(emb[:, 0, :], w1_ref[...],
                preferred_element_type=jnp.float32) + w1b[None, :]
    l = jnp.dot(emb[:, 1, :], w2_ref[...],
                preferred_element_type=jnp.float32) + w2b[None, :]
    out_ref[...] = jnp.concatenate(
        [f[:, None, :], l[:, None, :], emb[:, 2:R, :]], axis=1)


def kernel(nodes, first_node_idx, last_node_idx, candidate_indices,
           W_coord, b_coord, W1_w, W1_b, W2_w, W2_b):
    x = nodes[:, :, 0]
    y = nodes[:, :, 1]
    idx = jnp.concatenate(
        [first_node_idx[:, None], last_node_idx[:, None],
         jnp.clip(candidate_indices, 0, None)], axis=1).astype(jnp.int32)
    idx = jnp.pad(idx, ((0, 0), (0, RPAD - R)))
    wrows = jnp.zeros((8, D), jnp.float32)
    wrows = wrows.at[0].set(W_coord[0]).at[1].set(W_coord[1])
    wrows = wrows.at[2].set(b_coord).at[3].set(W1_b).at[4].set(W2_b)

    grid = (B // TB,)
    out = pl.pallas_call(
        _kernel,
        grid=grid,
        in_specs=[
            pl.BlockSpec((TB, RPAD), lambda i: (i, 0)),
            pl.BlockSpec((TB, N), lambda i: (i, 0)),
            pl.BlockSpec((TB, N), lambda i: (i, 0)),
            pl.BlockSpec((8, D), lambda i: (0, 0)),
            pl.BlockSpec((D, D), lambda i: (0, 0)),
            pl.BlockSpec((D, D), lambda i: (0, 0)),
        ],
        out_specs=pl.BlockSpec((TB, R, D), lambda i: (i, 0, 0)),
        out_shape=jax.ShapeDtypeStruct((B, R, D), jnp.float32),
    )(idx, x, y, wrows, W1_w, W2_w)
    return out
